# Initial kernel scaffold; baseline (speedup 1.0000x reference)
import jax
import jax.numpy as jnp
from jax.experimental import pallas as pl


def kernel(node_features, edge_features, senders, receivers, ew1, eb1, ew2, eb2, ew3, eb3, eg, ebt, nw1, nb1, nw2, nb2, nw3, nb3, ng, nbt):
    raise NotImplementedError("write your pallas kernel here")


# speedup vs baseline: 3.1205x; 3.1205x over previous
import functools

import jax
import jax.numpy as jnp
from jax import lax
from jax.experimental import pallas as pl
from jax.experimental.pallas import tpu as pltpu
from jax.experimental.pallas import tpu_sc as plsc

N = 10000
E = 320000
D = 128

NC = 2
NS = 16
NW = NC * NS
EPW = E // NW
C = 80
ROWS_PER_TILE = 640


def _project_kernel(nf_ref, ws_ref, wr_ref, ps_ref, pr_ref):
    x = nf_ref[:]
    ps_ref[:] = jnp.dot(x, ws_ref[:], preferred_element_type=jnp.float32)
    pr_ref[:] = jnp.dot(x, wr_ref[:], preferred_element_type=jnp.float32)


def _project(nf, w1s, w1r):
    BN = 1000
    grid = (N // BN,)
    return pl.pallas_call(
        _project_kernel,
        grid=grid,
        in_specs=[
            pl.BlockSpec((BN, D), lambda i: (i, 0)),
            pl.BlockSpec((D, D), lambda i: (0, 0)),
            pl.BlockSpec((D, D), lambda i: (0, 0)),
        ],
        out_specs=[
            pl.BlockSpec((BN, D), lambda i: (i, 0)),
            pl.BlockSpec((BN, D), lambda i: (i, 0)),
        ],
        out_shape=[
            jax.ShapeDtypeStruct((N, D), jnp.float32),
            jax.ShapeDtypeStruct((N, D), jnp.float32),
        ],
    )(nf, w1s, w1r)


def _edge_mlp_kernel(gs_ref, gr_ref, ef_ref, w1_ref, b1_ref, w2_ref, b2_ref,
                     w3_ref, b3_ref, g_ref, bt_ref, ln_ref, out_ref):
    ef = ef_ref[:]
    h = gs_ref[:] + gr_ref[:] + b1_ref[:] + jnp.dot(
        ef, w1_ref[:], preferred_element_type=jnp.float32)
    h = jnp.maximum(h, 0.0)
    h = jnp.dot(h, w2_ref[:], preferred_element_type=jnp.float32) + b2_ref[:]
    h = jnp.maximum(h, 0.0)
    h = jnp.dot(h, w3_ref[:], preferred_element_type=jnp.float32) + b3_ref[:]
    mu = jnp.mean(h, axis=1, keepdims=True)
    d = h - mu
    var = jnp.mean(d * d, axis=1, keepdims=True)
    y = d * lax.rsqrt(var + 1e-5) * g_ref[:] + bt_ref[:]
    ln_ref[:] = y
    out_ref[:] = y + ef


def _edge_mlp(gs, gr, ef, w1e, b1, w2, b2, w3, b3, g, bt):
    BE = 2000
    grid = (E // BE,)
    blk = lambda i: (i, 0)
    wspec = pl.BlockSpec((D, D), lambda i: (0, 0))
    vspec = pl.BlockSpec((1, D), lambda i: (0, 0))
    return pl.pallas_call(
        _edge_mlp_kernel,
        grid=grid,
        in_specs=[
            pl.BlockSpec((BE, D), blk),
            pl.BlockSpec((BE, D), blk),
            pl.BlockSpec((BE, D), blk),
            wspec, vspec, wspec, vspec, wspec, vspec, vspec, vspec,
        ],
        out_specs=[pl.BlockSpec((BE, D), blk), pl.BlockSpec((BE, D), blk)],
        out_shape=[
            jax.ShapeDtypeStruct((E, D), jnp.float32),
            jax.ShapeDtypeStruct((E, D), jnp.float32),
        ],
    )(gs, gr, ef, w1e, b1, w2, b2, w3, b3, g, bt)


def _node_mlp_kernel(nf_ref, p0_ref, p1_ref, w1n_ref, w1a_ref, b1_ref,
                     w2_ref, b2_ref, w3_ref, b3_ref, g_ref, bt_ref, out_ref):
    nf = nf_ref[:]
    agg = p0_ref[:] + p1_ref[:]
    h = (jnp.dot(nf, w1n_ref[:], preferred_element_type=jnp.float32)
         + jnp.dot(agg, w1a_ref[:], preferred_element_type=jnp.float32)
         + b1_ref[:])
    h = jnp.maximum(h, 0.0)
    h = jnp.dot(h, w2_ref[:], preferred_element_type=jnp.float32) + b2_ref[:]
    h = jnp.maximum(h, 0.0)
    h = jnp.dot(h, w3_ref[:], preferred_element_type=jnp.float32) + b3_ref[:]
    mu = jnp.mean(h, axis=1, keepdims=True)
    d = h - mu
    var = jnp.mean(d * d, axis=1, keepdims=True)
    y = d * lax.rsqrt(var + 1e-5) * g_ref[:] + bt_ref[:]
    out_ref[:] = y + nf


def _node_mlp(nf, p0, p1, w1n, w1a, b1, w2, b2, w3, b3, g, bt):
    BN = 1000
    grid = (N // BN,)
    blk = lambda i: (i, 0)
    wspec = pl.BlockSpec((D, D), lambda i: (0, 0))
    vspec = pl.BlockSpec((1, D), lambda i: (0, 0))
    return pl.pallas_call(
        _node_mlp_kernel,
        grid=grid,
        in_specs=[
            pl.BlockSpec((BN, D), blk),
            pl.BlockSpec((BN, D), blk),
            pl.BlockSpec((BN, D), blk),
            wspec, wspec, vspec, wspec, vspec, wspec, vspec, vspec, vspec,
        ],
        out_specs=pl.BlockSpec((BN, D), blk),
        out_shape=jax.ShapeDtypeStruct((N, D), jnp.float32),
    )(nf, p0, p1, w1n, w1a, b1, w2, b2, w3, b3, g, bt)


def _gather(ps, pr, snd, rcv):
    mesh = plsc.VectorSubcoreMesh(core_axis_name="c", subcore_axis_name="s")

    @functools.partial(
        pl.kernel,
        mesh=mesh,
        out_type=(jax.ShapeDtypeStruct((E, D), jnp.float32),
                  jax.ShapeDtypeStruct((E, D), jnp.float32)),
        scratch_types=[
            pltpu.VMEM((C,), jnp.int32),
            pltpu.VMEM((C,), jnp.int32),
            pltpu.VMEM((C, D), jnp.float32),
            pltpu.VMEM((C, D), jnp.float32),
            pltpu.SemaphoreType.DMA,
            pltpu.SemaphoreType.DMA,
        ],
    )
    def k(ps_h, pr_h, snd_h, rcv_h, gs_h, gr_h, sidx, ridx, srow, rrow,
          sem_a, sem_b):
        c = lax.axis_index("c")
        s = lax.axis_index("s")
        wid = s * NC + c
        base = wid * EPW

        def body(i, carry):
            off = base + i * C
            pltpu.sync_copy(snd_h.at[pl.ds(off, C)], sidx)
            pltpu.sync_copy(rcv_h.at[pl.ds(off, C)], ridx)
            ca = pltpu.async_copy(ps_h.at[sidx], srow, sem_a)
            cb = pltpu.async_copy(pr_h.at[ridx], rrow, sem_b)
            ca.wait()
            cb.wait()
            pltpu.sync_copy(srow, gs_h.at[pl.ds(off, C)])
            pltpu.sync_copy(rrow, gr_h.at[pl.ds(off, C)])
            return carry

        lax.fori_loop(0, EPW // C, body, 0)

    return k(ps, pr, snd, rcv)


def _scatter(new_e_ln, rcv, zeros):
    mesh = plsc.VectorSubcoreMesh(core_axis_name="c", subcore_axis_name="s")
    epc = E // NC

    @functools.partial(
        pl.kernel,
        mesh=mesh,
        out_type=jax.ShapeDtypeStruct((NC, N, D), jnp.float32),
        scratch_types=[
            pltpu.VMEM((C,), jnp.int32),
            pltpu.VMEM((C, D), jnp.float32),
            pltpu.VMEM_SHARED((N, D), jnp.float32),
        ],
    )
    def k(e_h, rcv_h, z_h, out_h, idx, rows, acc):
        c = lax.axis_index("c")
        s = lax.axis_index("s")
        r0 = s * ROWS_PER_TILE
        @pl.when(s < NS - 1)
        def _():
            pltpu.sync_copy(z_h.at[pl.ds(r0, ROWS_PER_TILE)],
                            acc.at[pl.ds(r0, ROWS_PER_TILE)])

        @pl.when(s == NS - 1)
        def _():
            pltpu.sync_copy(z_h.at[pl.ds(r0, N - (NS - 1) * ROWS_PER_TILE)],
                            acc.at[pl.ds(r0, N - (NS - 1) * ROWS_PER_TILE)])

        plsc.subcore_barrier()

        base = c * epc + s * (epc // NS)

        def body(i, carry):
            off = base + i * C
            pltpu.sync_copy(rcv_h.at[pl.ds(off, C)], idx)
            pltpu.sync_copy(e_h.at[pl.ds(off, C)], rows)
            pltpu.sync_copy(rows, acc.at[idx], add=True)
            return carry

        lax.fori_loop(0, (epc // NS) // C, body, 0)
        plsc.subcore_barrier()

        @pl.when(s < NS - 1)
        def _():
            pltpu.sync_copy(acc.at[pl.ds(r0, ROWS_PER_TILE)],
                            out_h.at[c, pl.ds(r0, ROWS_PER_TILE)])

        @pl.when(s == NS - 1)
        def _():
            pltpu.sync_copy(acc.at[pl.ds(r0, N - (NS - 1) * ROWS_PER_TILE)],
                            out_h.at[c, pl.ds(r0, N - (NS - 1) * ROWS_PER_TILE)])

    return k(new_e_ln, rcv, zeros)


def kernel(node_features, edge_features, senders, receivers,
           ew1, eb1, ew2, eb2, ew3, eb3, eg, ebt,
           nw1, nb1, nw2, nb2, nw3, nb3, ng, nbt):
    w1s, w1r, w1e = ew1[:D], ew1[D:2 * D], ew1[2 * D:]
    w1n, w1a = nw1[:D], nw1[D:]
    eb1r = eb1.reshape(1, D)
    eb2r = eb2.reshape(1, D)
    eb3r = eb3.reshape(1, D)
    egr = eg.reshape(1, D)
    ebtr = ebt.reshape(1, D)
    nb1r = nb1.reshape(1, D)
    nb2r = nb2.reshape(1, D)
    nb3r = nb3.reshape(1, D)
    ngr = ng.reshape(1, D)
    nbtr = nbt.reshape(1, D)

    ps, pr = _project(node_features, w1s, w1r)
    gs, gr = _gather(ps, pr, senders, receivers)
    new_e_ln, new_e = _edge_mlp(gs, gr, edge_features, w1e, eb1r, ew2, eb2r,
                                ew3, eb3r, egr, ebtr)
    zeros = jnp.zeros((N, D), jnp.float32)
    parts = _scatter(new_e_ln, receivers, zeros)
    new_n = _node_mlp(node_features, parts[0], parts[1], w1n, w1a, nb1r,
                      nw2, nb2r, nw3, nb3r, ngr, nbtr)
    return (new_n, new_e)


# speedup vs baseline: 4.1737x; 1.3375x over previous
import functools

import jax
import jax.numpy as jnp
from jax import lax
from jax.experimental import pallas as pl
from jax.experimental.pallas import tpu as pltpu
from jax.experimental.pallas import tpu_sc as plsc

N = 10000
E = 320000
D = 128

NC = 2
NS = 16
NW = NC * NS
EPW = E // NW
C = 80
ROWS_PER_TILE = 640


def _project_kernel(nf_ref, ws_ref, wr_ref, ps_ref, pr_ref):
    x = nf_ref[:]
    ps_ref[:] = jnp.dot(x, ws_ref[:], preferred_element_type=jnp.float32)
    pr_ref[:] = jnp.dot(x, wr_ref[:], preferred_element_type=jnp.float32)


def _project(nf, w1s, w1r):
    BN = 1000
    grid = (N // BN,)
    return pl.pallas_call(
        _project_kernel,
        grid=grid,
        in_specs=[
            pl.BlockSpec((BN, D), lambda i: (i, 0)),
            pl.BlockSpec((D, D), lambda i: (0, 0)),
            pl.BlockSpec((D, D), lambda i: (0, 0)),
        ],
        out_specs=[
            pl.BlockSpec((BN, D), lambda i: (i, 0)),
            pl.BlockSpec((BN, D), lambda i: (i, 0)),
        ],
        out_shape=[
            jax.ShapeDtypeStruct((N, D), jnp.float32),
            jax.ShapeDtypeStruct((N, D), jnp.float32),
        ],
    )(nf, w1s, w1r)


def _edge_mlp_kernel(gs_ref, gr_ref, ef_ref, w1_ref, b1_ref, w2_ref, b2_ref,
                     w3_ref, b3_ref, g_ref, bt_ref, ln_ref, out_ref):
    ef = ef_ref[:]
    h = gs_ref[:] + gr_ref[:] + b1_ref[:] + jnp.dot(
        ef, w1_ref[:], preferred_element_type=jnp.float32)
    h = jnp.maximum(h, 0.0)
    h = jnp.dot(h, w2_ref[:], preferred_element_type=jnp.float32) + b2_ref[:]
    h = jnp.maximum(h, 0.0)
    h = jnp.dot(h, w3_ref[:], preferred_element_type=jnp.float32) + b3_ref[:]
    mu = jnp.mean(h, axis=1, keepdims=True)
    d = h - mu
    var = jnp.mean(d * d, axis=1, keepdims=True)
    y = d * lax.rsqrt(var + 1e-5) * g_ref[:] + bt_ref[:]
    ln_ref[:] = y
    out_ref[:] = y + ef


def _edge_mlp(gs, gr, ef, w1e, b1, w2, b2, w3, b3, g, bt):
    BE = 2000
    grid = (E // BE,)
    blk = lambda i: (i, 0)
    wspec = pl.BlockSpec((D, D), lambda i: (0, 0))
    vspec = pl.BlockSpec((1, D), lambda i: (0, 0))
    return pl.pallas_call(
        _edge_mlp_kernel,
        grid=grid,
        in_specs=[
            pl.BlockSpec((BE, D), blk),
            pl.BlockSpec((BE, D), blk),
            pl.BlockSpec((BE, D), blk),
            wspec, vspec, wspec, vspec, wspec, vspec, vspec, vspec,
        ],
        out_specs=[pl.BlockSpec((BE, D), blk), pl.BlockSpec((BE, D), blk)],
        out_shape=[
            jax.ShapeDtypeStruct((E, D), jnp.float32),
            jax.ShapeDtypeStruct((E, D), jnp.float32),
        ],
    )(gs, gr, ef, w1e, b1, w2, b2, w3, b3, g, bt)


def _node_mlp_kernel(nf_ref, p0_ref, p1_ref, w1n_ref, w1a_ref, b1_ref,
                     w2_ref, b2_ref, w3_ref, b3_ref, g_ref, bt_ref, out_ref):
    nf = nf_ref[:]
    agg = p0_ref[:] + p1_ref[:]
    h = (jnp.dot(nf, w1n_ref[:], preferred_element_type=jnp.float32)
         + jnp.dot(agg, w1a_ref[:], preferred_element_type=jnp.float32)
         + b1_ref[:])
    h = jnp.maximum(h, 0.0)
    h = jnp.dot(h, w2_ref[:], preferred_element_type=jnp.float32) + b2_ref[:]
    h = jnp.maximum(h, 0.0)
    h = jnp.dot(h, w3_ref[:], preferred_element_type=jnp.float32) + b3_ref[:]
    mu = jnp.mean(h, axis=1, keepdims=True)
    d = h - mu
    var = jnp.mean(d * d, axis=1, keepdims=True)
    y = d * lax.rsqrt(var + 1e-5) * g_ref[:] + bt_ref[:]
    out_ref[:] = y + nf


def _node_mlp(nf, p0, p1, w1n, w1a, b1, w2, b2, w3, b3, g, bt):
    BN = 1000
    grid = (N // BN,)
    blk = lambda i: (i, 0)
    wspec = pl.BlockSpec((D, D), lambda i: (0, 0))
    vspec = pl.BlockSpec((1, D), lambda i: (0, 0))
    return pl.pallas_call(
        _node_mlp_kernel,
        grid=grid,
        in_specs=[
            pl.BlockSpec((BN, D), blk),
            pl.BlockSpec((BN, D), blk),
            pl.BlockSpec((BN, D), blk),
            wspec, wspec, vspec, wspec, vspec, wspec, vspec, vspec, vspec,
        ],
        out_specs=pl.BlockSpec((BN, D), blk),
        out_shape=jax.ShapeDtypeStruct((N, D), jnp.float32),
    )(nf, p0, p1, w1n, w1a, b1, w2, b2, w3, b3, g, bt)


def _gather(ps, pr, snd, rcv):
    mesh = plsc.VectorSubcoreMesh(core_axis_name="c", subcore_axis_name="s")
    epc = E // NS

    @functools.partial(
        pl.kernel,
        mesh=mesh,
        out_type=(jax.ShapeDtypeStruct((E, D), jnp.float32),
                  jax.ShapeDtypeStruct((E, D), jnp.float32)),
        scratch_types=[
            pltpu.VMEM((C,), jnp.int32),
            pltpu.VMEM((C,), jnp.int32),
            pltpu.VMEM((C, D), jnp.float32),
            pltpu.VMEM((C, D), jnp.float32),
            pltpu.VMEM_SHARED((N, D), jnp.float32),
            pltpu.SemaphoreType.DMA,
            pltpu.SemaphoreType.DMA,
        ],
    )
    def k(ps_h, pr_h, snd_h, rcv_h, gs_h, gr_h, idxa, idxb, row0, row1, tab,
          sem0, sem1):
        c = lax.axis_index("c")
        s = lax.axis_index("s")

        def run(tab_h, idx_h, out_h):
            r0 = s * ROWS_PER_TILE

            @pl.when(s < NS - 1)
            def _():
                pltpu.sync_copy(tab_h.at[pl.ds(r0, ROWS_PER_TILE)],
                                tab.at[pl.ds(r0, ROWS_PER_TILE)])

            @pl.when(s == NS - 1)
            def _():
                nrem = N - (NS - 1) * ROWS_PER_TILE
                pltpu.sync_copy(tab_h.at[pl.ds(r0, nrem)],
                                tab.at[pl.ds(r0, nrem)])

            plsc.subcore_barrier()

            base = s * epc

            def body(i, carry):
                off = base + 2 * i * C
                pltpu.sync_copy(idx_h.at[pl.ds(off, C)], idxa)
                ca = pltpu.async_copy(tab.at[idxa], row0, sem0)
                pltpu.sync_copy(idx_h.at[pl.ds(off + C, C)], idxb)
                cb = pltpu.async_copy(tab.at[idxb], row1, sem1)
                ca.wait()
                pltpu.sync_copy(row0, out_h.at[pl.ds(off, C)])
                cb.wait()
                pltpu.sync_copy(row1, out_h.at[pl.ds(off + C, C)])
                return carry

            lax.fori_loop(0, epc // (2 * C), body, 0)

        @pl.when(c == 0)
        def _():
            run(ps_h, snd_h, gs_h)

        @pl.when(c == 1)
        def _():
            run(pr_h, rcv_h, gr_h)

    return k(ps, pr, snd, rcv)


def _scatter(new_e_ln, rcv, zeros):
    mesh = plsc.VectorSubcoreMesh(core_axis_name="c", subcore_axis_name="s")
    epc = E // NC

    @functools.partial(
        pl.kernel,
        mesh=mesh,
        out_type=jax.ShapeDtypeStruct((NC, N, D), jnp.float32),
        scratch_types=[
            pltpu.VMEM((C,), jnp.int32),
            pltpu.VMEM((C,), jnp.int32),
            pltpu.VMEM((C, D), jnp.float32),
            pltpu.VMEM((C, D), jnp.float32),
            pltpu.VMEM_SHARED((N, D), jnp.float32),
            pltpu.SemaphoreType.DMA,
            pltpu.SemaphoreType.DMA,
        ],
    )
    def k(e_h, rcv_h, z_h, out_h, idxa, idxb, rows0, rows1, acc, sem0, sem1):
        c = lax.axis_index("c")
        s = lax.axis_index("s")
        r0 = s * ROWS_PER_TILE
        @pl.when(s < NS - 1)
        def _():
            pltpu.sync_copy(z_h.at[pl.ds(r0, ROWS_PER_TILE)],
                            acc.at[pl.ds(r0, ROWS_PER_TILE)])

        @pl.when(s == NS - 1)
        def _():
            pltpu.sync_copy(z_h.at[pl.ds(r0, N - (NS - 1) * ROWS_PER_TILE)],
                            acc.at[pl.ds(r0, N - (NS - 1) * ROWS_PER_TILE)])

        plsc.subcore_barrier()

        base = c * epc + s * (epc // NS)

        def body(i, carry):
            off = base + 2 * i * C
            pltpu.sync_copy(rcv_h.at[pl.ds(off, C)], idxa)
            ca = pltpu.async_copy(e_h.at[pl.ds(off, C)], rows0, sem0)
            pltpu.sync_copy(rcv_h.at[pl.ds(off + C, C)], idxb)
            cb = pltpu.async_copy(e_h.at[pl.ds(off + C, C)], rows1, sem1)
            ca.wait()
            pltpu.sync_copy(rows0, acc.at[idxa], add=True)
            cb.wait()
            pltpu.sync_copy(rows1, acc.at[idxb], add=True)
            return carry

        ept = epc // NS
        lax.fori_loop(0, ept // (2 * C), body, 0)
        if ept % (2 * C) != 0:
            toff = base + (ept // (2 * C)) * 2 * C
            pltpu.sync_copy(rcv_h.at[pl.ds(toff, C)], idxa)
            pltpu.sync_copy(e_h.at[pl.ds(toff, C)], rows0)
            pltpu.sync_copy(rows0, acc.at[idxa], add=True)
        plsc.subcore_barrier()

        @pl.when(s < NS - 1)
        def _():
            pltpu.sync_copy(acc.at[pl.ds(r0, ROWS_PER_TILE)],
                            out_h.at[c, pl.ds(r0, ROWS_PER_TILE)])

        @pl.when(s == NS - 1)
        def _():
            pltpu.sync_copy(acc.at[pl.ds(r0, N - (NS - 1) * ROWS_PER_TILE)],
                            out_h.at[c, pl.ds(r0, N - (NS - 1) * ROWS_PER_TILE)])

    return k(new_e_ln, rcv, zeros)


def kernel(node_features, edge_features, senders, receivers,
           ew1, eb1, ew2, eb2, ew3, eb3, eg, ebt,
           nw1, nb1, nw2, nb2, nw3, nb3, ng, nbt):
    w1s, w1r, w1e = ew1[:D], ew1[D:2 * D], ew1[2 * D:]
    w1n, w1a = nw1[:D], nw1[D:]
    eb1r = eb1.reshape(1, D)
    eb2r = eb2.reshape(1, D)
    eb3r = eb3.reshape(1, D)
    egr = eg.reshape(1, D)
    ebtr = ebt.reshape(1, D)
    nb1r = nb1.reshape(1, D)
    nb2r = nb2.reshape(1, D)
    nb3r = nb3.reshape(1, D)
    ngr = ng.reshape(1, D)
    nbtr = nbt.reshape(1, D)

    ps, pr = _project(node_features, w1s, w1r)
    gs, gr = _gather(ps, pr, senders, receivers)
    new_e_ln, new_e = _edge_mlp(gs, gr, edge_features, w1e, eb1r, ew2, eb2r,
                                ew3, eb3r, egr, ebtr)
    zeros = jnp.zeros((N, D), jnp.float32)
    parts = _scatter(new_e_ln, receivers, zeros)
    new_n = _node_mlp(node_features, parts[0], parts[1], w1n, w1a, nb1r,
                      nw2, nb2r, nw3, nb3r, ngr, nbtr)
    return (new_n, new_e)


# speedup vs baseline: 4.5414x; 1.0881x over previous
import functools

import jax
import jax.numpy as jnp
from jax import lax
from jax.experimental import pallas as pl
from jax.experimental.pallas import tpu as pltpu
from jax.experimental.pallas import tpu_sc as plsc

N = 10000
E = 320000
D = 128

NC = 2
NS = 16
C = 80
ROWS_PER_TILE = 640

K = 5
CE = E // K


def _project_kernel(nf_ref, ws_ref, wr_ref, ps_ref, pr_ref):
    x = nf_ref[:]
    ps_ref[:] = jnp.dot(x, ws_ref[:], preferred_element_type=jnp.float32)
    pr_ref[:] = jnp.dot(x, wr_ref[:], preferred_element_type=jnp.float32)


def _project(nf, w1s, w1r):
    BN = 1000
    grid = (N // BN,)
    return pl.pallas_call(
        _project_kernel,
        grid=grid,
        in_specs=[
            pl.BlockSpec((BN, D), lambda i: (i, 0)),
            pl.BlockSpec((D, D), lambda i: (0, 0)),
            pl.BlockSpec((D, D), lambda i: (0, 0)),
        ],
        out_specs=[
            pl.BlockSpec((BN, D), lambda i: (i, 0)),
            pl.BlockSpec((BN, D), lambda i: (i, 0)),
        ],
        out_shape=[
            jax.ShapeDtypeStruct((N, D), jnp.float32),
            jax.ShapeDtypeStruct((N, D), jnp.float32),
        ],
    )(nf, w1s, w1r)


def _edge_mlp_kernel(gs_ref, gr_ref, ef_ref, w1_ref, b1_ref, w2_ref, b2_ref,
                     w3_ref, b3_ref, g_ref, bt_ref, ln_ref, out_ref):
    ef = ef_ref[:]
    h = gs_ref[:] + gr_ref[:] + b1_ref[:] + jnp.dot(
        ef, w1_ref[:], preferred_element_type=jnp.float32)
    h = jnp.maximum(h, 0.0)
    h = jnp.dot(h, w2_ref[:], preferred_element_type=jnp.float32) + b2_ref[:]
    h = jnp.maximum(h, 0.0)
    h = jnp.dot(h, w3_ref[:], preferred_element_type=jnp.float32) + b3_ref[:]
    mu = jnp.mean(h, axis=1, keepdims=True)
    d = h - mu
    var = jnp.mean(d * d, axis=1, keepdims=True)
    y = d * lax.rsqrt(var + 1e-5) * g_ref[:] + bt_ref[:]
    ln_ref[:] = y
    out_ref[:] = y + ef


def _edge_mlp_chunk(k, gs, gr, ef, w1e, b1, w2, b2, w3, b3, g, bt, e_acc):
    BE = 2000
    nblk = CE // BE
    koff = k * nblk
    blk = lambda i: (i, 0)
    out_blk = lambda i: (i + koff, 0)
    wspec = pl.BlockSpec((D, D), lambda i: (0, 0))
    vspec = pl.BlockSpec((1, D), lambda i: (0, 0))
    in_specs = [
        pl.BlockSpec((BE, D), blk),
        pl.BlockSpec((BE, D), blk),
        pl.BlockSpec((BE, D), out_blk),
        wspec, vspec, wspec, vspec, wspec, vspec, vspec, vspec,
    ]
    args = [gs, gr, ef, w1e, b1, w2, b2, w3, b3, g, bt]
    kwargs = {}
    kfn = _edge_mlp_kernel
    if e_acc is not None:
        in_specs.append(pl.BlockSpec(memory_space=pl.ANY))
        args.append(e_acc)
        kwargs["input_output_aliases"] = {11: 1}
        kfn = lambda *refs: _edge_mlp_kernel(*refs[:11], *refs[12:])
    return pl.pallas_call(
        kfn,
        grid=(nblk,),
        in_specs=in_specs,
        out_specs=[pl.BlockSpec((BE, D), blk), pl.BlockSpec((BE, D), out_blk)],
        out_shape=[
            jax.ShapeDtypeStruct((CE, D), jnp.float32),
            jax.ShapeDtypeStruct((E, D), jnp.float32),
        ],
        **kwargs,
    )(*args)


def _node_mlp_kernel(nf_ref, p0_ref, p1_ref, p2_ref, p3_ref, p4_ref,
                     w1n_ref, w1a_ref, b1_ref,
                     w2_ref, b2_ref, w3_ref, b3_ref, g_ref, bt_ref, out_ref):
    nf = nf_ref[:]
    agg = (p0_ref[0] + p0_ref[1] + p1_ref[0] + p1_ref[1]
           + p2_ref[0] + p2_ref[1] + p3_ref[0] + p3_ref[1]
           + p4_ref[0] + p4_ref[1])
    h = (jnp.dot(nf, w1n_ref[:], preferred_element_type=jnp.float32)
         + jnp.dot(agg, w1a_ref[:], preferred_element_type=jnp.float32)
         + b1_ref[:])
    h = jnp.maximum(h, 0.0)
    h = jnp.dot(h, w2_ref[:], preferred_element_type=jnp.float32) + b2_ref[:]
    h = jnp.maximum(h, 0.0)
    h = jnp.dot(h, w3_ref[:], preferred_element_type=jnp.float32) + b3_ref[:]
    mu = jnp.mean(h, axis=1, keepdims=True)
    d = h - mu
    var = jnp.mean(d * d, axis=1, keepdims=True)
    y = d * lax.rsqrt(var + 1e-5) * g_ref[:] + bt_ref[:]
    out_ref[:] = y + nf


def _node_mlp(nf, parts, w1n, w1a, b1, w2, b2, w3, b3, g, bt):
    BN = 1000
    grid = (N // BN,)
    blk = lambda i: (i, 0)
    pspec = pl.BlockSpec((NC, BN, D), lambda i: (0, i, 0))
    wspec = pl.BlockSpec((D, D), lambda i: (0, 0))
    vspec = pl.BlockSpec((1, D), lambda i: (0, 0))
    return pl.pallas_call(
        _node_mlp_kernel,
        grid=grid,
        in_specs=[
            pl.BlockSpec((BN, D), blk),
            pspec, pspec, pspec, pspec, pspec,
            wspec, wspec, vspec, wspec, vspec, wspec, vspec, vspec, vspec,
        ],
        out_specs=pl.BlockSpec((BN, D), blk),
        out_shape=jax.ShapeDtypeStruct((N, D), jnp.float32),
    )(nf, *parts, w1n, w1a, b1, w2, b2, w3, b3, g, bt)


def _gather(ps, pr, snd, rcv):
    mesh = plsc.VectorSubcoreMesh(core_axis_name="c", subcore_axis_name="s")
    epc = CE // NS

    @functools.partial(
        pl.kernel,
        mesh=mesh,
        out_type=(jax.ShapeDtypeStruct((CE, D), jnp.float32),
                  jax.ShapeDtypeStruct((CE, D), jnp.float32)),
        scratch_types=[
            pltpu.VMEM((C,), jnp.int32),
            pltpu.VMEM((C,), jnp.int32),
            pltpu.VMEM((C, D), jnp.float32),
            pltpu.VMEM((C, D), jnp.float32),
            pltpu.VMEM_SHARED((N, D), jnp.float32),
            pltpu.SemaphoreType.DMA,
            pltpu.SemaphoreType.DMA,
        ],
    )
    def k(ps_h, pr_h, snd_h, rcv_h, gs_h, gr_h, idxa, idxb, row0, row1, tab,
          sem0, sem1):
        c = lax.axis_index("c")
        s = lax.axis_index("s")

        def run(tab_h, idx_h, out_h):
            r0 = s * ROWS_PER_TILE

            @pl.when(s < NS - 1)
            def _():
                pltpu.sync_copy(tab_h.at[pl.ds(r0, ROWS_PER_TILE)],
                                tab.at[pl.ds(r0, ROWS_PER_TILE)])

            @pl.when(s == NS - 1)
            def _():
                nrem = N - (NS - 1) * ROWS_PER_TILE
                pltpu.sync_copy(tab_h.at[pl.ds(r0, nrem)],
                                tab.at[pl.ds(r0, nrem)])

            plsc.subcore_barrier()

            base = s * epc

            def body(i, carry):
                off = base + 2 * i * C
                pltpu.sync_copy(idx_h.at[pl.ds(off, C)], idxa)
                ca = pltpu.async_copy(tab.at[idxa], row0, sem0)
                pltpu.sync_copy(idx_h.at[pl.ds(off + C, C)], idxb)
                cb = pltpu.async_copy(tab.at[idxb], row1, sem1)
                ca.wait()
                pltpu.sync_copy(row0, out_h.at[pl.ds(off, C)])
                cb.wait()
                pltpu.sync_copy(row1, out_h.at[pl.ds(off + C, C)])
                return carry

            lax.fori_loop(0, epc // (2 * C), body, 0)

        @pl.when(c == 0)
        def _():
            run(ps_h, snd_h, gs_h)

        @pl.when(c == 1)
        def _():
            run(pr_h, rcv_h, gr_h)

    return k(ps, pr, snd, rcv)


def _scatter(new_e_ln, rcv, zeros):
    mesh = plsc.VectorSubcoreMesh(core_axis_name="c", subcore_axis_name="s")
    epc = CE // NC

    @functools.partial(
        pl.kernel,
        mesh=mesh,
        out_type=jax.ShapeDtypeStruct((NC, N, D), jnp.float32),
        scratch_types=[
            pltpu.VMEM((C,), jnp.int32),
            pltpu.VMEM((C,), jnp.int32),
            pltpu.VMEM((C, D), jnp.float32),
            pltpu.VMEM((C, D), jnp.float32),
            pltpu.VMEM_SHARED((N, D), jnp.float32),
            pltpu.SemaphoreType.DMA,
            pltpu.SemaphoreType.DMA,
        ],
    )
    def k(e_h, rcv_h, z_h, out_h, idxa, idxb, rows0, rows1, acc, sem0, sem1):
        c = lax.axis_index("c")
        s = lax.axis_index("s")
        r0 = s * ROWS_PER_TILE
        @pl.when(s < NS - 1)
        def _():
            pltpu.sync_copy(z_h.at[pl.ds(r0, ROWS_PER_TILE)],
                            acc.at[pl.ds(r0, ROWS_PER_TILE)])

        @pl.when(s == NS - 1)
        def _():
            pltpu.sync_copy(z_h.at[pl.ds(r0, N - (NS - 1) * ROWS_PER_TILE)],
                            acc.at[pl.ds(r0, N - (NS - 1) * ROWS_PER_TILE)])

        plsc.subcore_barrier()

        ept = epc // NS
        base = c * epc + s * ept

        def body(i, carry):
            off = base + 2 * i * C
            pltpu.sync_copy(rcv_h.at[pl.ds(off, C)], idxa)
            ca = pltpu.async_copy(e_h.at[pl.ds(off, C)], rows0, sem0)
            pltpu.sync_copy(rcv_h.at[pl.ds(off + C, C)], idxb)
            cb = pltpu.async_copy(e_h.at[pl.ds(off + C, C)], rows1, sem1)
            ca.wait()
            pltpu.sync_copy(rows0, acc.at[idxa], add=True)
            cb.wait()
            pltpu.sync_copy(rows1, acc.at[idxb], add=True)
            return carry

        lax.fori_loop(0, ept // (2 * C), body, 0)
        if ept % (2 * C) != 0:
            toff = base + (ept // (2 * C)) * 2 * C
            pltpu.sync_copy(rcv_h.at[pl.ds(toff, C)], idxa)
            pltpu.sync_copy(e_h.at[pl.ds(toff, C)], rows0)
            pltpu.sync_copy(rows0, acc.at[idxa], add=True)
        plsc.subcore_barrier()

        @pl.when(s < NS - 1)
        def _():
            pltpu.sync_copy(acc.at[pl.ds(r0, ROWS_PER_TILE)],
                            out_h.at[c, pl.ds(r0, ROWS_PER_TILE)])

        @pl.when(s == NS - 1)
        def _():
            pltpu.sync_copy(acc.at[pl.ds(r0, N - (NS - 1) * ROWS_PER_TILE)],
                            out_h.at[c, pl.ds(r0, N - (NS - 1) * ROWS_PER_TILE)])

    return k(new_e_ln, rcv, zeros)


def kernel(node_features, edge_features, senders, receivers,
           ew1, eb1, ew2, eb2, ew3, eb3, eg, ebt,
           nw1, nb1, nw2, nb2, nw3, nb3, ng, nbt):
    w1s, w1r, w1e = ew1[:D], ew1[D:2 * D], ew1[2 * D:]
    w1n, w1a = nw1[:D], nw1[D:]
    eb1r = eb1.reshape(1, D)
    eb2r = eb2.reshape(1, D)
    eb3r = eb3.reshape(1, D)
    egr = eg.reshape(1, D)
    ebtr = ebt.reshape(1, D)
    nb1r = nb1.reshape(1, D)
    nb2r = nb2.reshape(1, D)
    nb3r = nb3.reshape(1, D)
    ngr = ng.reshape(1, D)
    nbtr = nbt.reshape(1, D)

    ps, pr = _project(node_features, w1s, w1r)
    zeros = jnp.zeros((N, D), jnp.float32)

    parts = []
    e_acc = None
    for k in range(K):
        lo = k * CE
        snd_k = lax.slice(senders, (lo,), (lo + CE,))
        rcv_k = lax.slice(receivers, (lo,), (lo + CE,))
        gs_k, gr_k = _gather(ps, pr, snd_k, rcv_k)
        ln_k, e_acc = _edge_mlp_chunk(k, gs_k, gr_k, edge_features,
                                      w1e, eb1r, ew2, eb2r, ew3, eb3r,
                                      egr, ebtr, e_acc)
        parts.append(_scatter(ln_k, rcv_k, zeros))

    new_n = _node_mlp(node_features, parts, w1n, w1a, nb1r,
                      nw2, nb2r, nw3, nb3r, ngr, nbtr)
    return (new_n, e_acc)


# speedup vs baseline: 4.9329x; 1.0862x over previous
import functools

import jax
import jax.numpy as jnp
from jax import lax
from jax.experimental import pallas as pl
from jax.experimental.pallas import tpu as pltpu
from jax.experimental.pallas import tpu_sc as plsc

N = 10000
E = 320000
D = 128

NC = 2
NS = 16
C = 80
ROWS_PER_TILE = 640

K = 5
CE = E // K
DW = D // 2


def _project_kernel(nf_ref, ws_ref, wr_ref, ps_ref, pr_ref):
    x = nf_ref[:]
    ps_ref[:] = jnp.dot(x, ws_ref[:], preferred_element_type=jnp.float32)
    pr_ref[:] = jnp.dot(x, wr_ref[:], preferred_element_type=jnp.float32)


def _project(nf, w1s, w1r):
    BN = 1000
    grid = (N // BN,)
    return pl.pallas_call(
        _project_kernel,
        grid=grid,
        in_specs=[
            pl.BlockSpec((BN, D), lambda i: (i, 0)),
            pl.BlockSpec((D, D), lambda i: (0, 0)),
            pl.BlockSpec((D, D), lambda i: (0, 0)),
        ],
        out_specs=[
            pl.BlockSpec((BN, D), lambda i: (i, 0)),
            pl.BlockSpec((BN, D), lambda i: (i, 0)),
        ],
        out_shape=[
            jax.ShapeDtypeStruct((N, D), jnp.float32),
            jax.ShapeDtypeStruct((N, D), jnp.float32),
        ],
    )(nf, w1s, w1r)


def _edge_mlp_kernel(gs_ref, gr_ref, ef_ref, w1_ref, b1_ref, w2_ref, b2_ref,
                     w3_ref, b3_ref, g_ref, bt_ref, ln_ref, out_ref):
    ef = ef_ref[:]
    h = (gs_ref[:].astype(jnp.float32) + gr_ref[:].astype(jnp.float32)
         + b1_ref[:] + jnp.dot(ef, w1_ref[:],
                               preferred_element_type=jnp.float32))
    h = jnp.maximum(h, 0.0)
    h = jnp.dot(h, w2_ref[:], preferred_element_type=jnp.float32) + b2_ref[:]
    h = jnp.maximum(h, 0.0)
    h = jnp.dot(h, w3_ref[:], preferred_element_type=jnp.float32) + b3_ref[:]
    mu = jnp.mean(h, axis=1, keepdims=True)
    d = h - mu
    var = jnp.mean(d * d, axis=1, keepdims=True)
    y = d * lax.rsqrt(var + 1e-5) * g_ref[:] + bt_ref[:]
    ln_ref[:] = y
    out_ref[:] = y + ef


def _edge_mlp_chunk(k, gs, gr, ef, w1e, b1, w2, b2, w3, b3, g, bt, e_acc):
    BE = 2000
    nblk = CE // BE
    koff = k * nblk
    blk = lambda i: (i, 0)
    out_blk = lambda i: (i + koff, 0)
    wspec = pl.BlockSpec((D, D), lambda i: (0, 0))
    vspec = pl.BlockSpec((1, D), lambda i: (0, 0))
    in_specs = [
        pl.BlockSpec((BE, D), blk),
        pl.BlockSpec((BE, D), blk),
        pl.BlockSpec((BE, D), out_blk),
        wspec, vspec, wspec, vspec, wspec, vspec, vspec, vspec,
    ]
    args = [gs, gr, ef, w1e, b1, w2, b2, w3, b3, g, bt]
    kwargs = {}
    kfn = _edge_mlp_kernel
    if e_acc is not None:
        in_specs.append(pl.BlockSpec(memory_space=pl.ANY))
        args.append(e_acc)
        kwargs["input_output_aliases"] = {11: 1}
        kfn = lambda *refs: _edge_mlp_kernel(*refs[:11], *refs[12:])
    return pl.pallas_call(
        kfn,
        grid=(nblk,),
        in_specs=in_specs,
        out_specs=[pl.BlockSpec((BE, D), blk), pl.BlockSpec((BE, D), out_blk)],
        out_shape=[
            jax.ShapeDtypeStruct((CE, D), jnp.float32),
            jax.ShapeDtypeStruct((E, D), jnp.float32),
        ],
        **kwargs,
    )(*args)


def _node_mlp_kernel(nf_ref, p0_ref, p1_ref,
                     w1n_ref, w1a_ref, b1_ref,
                     w2_ref, b2_ref, w3_ref, b3_ref, g_ref, bt_ref, out_ref):
    nf = nf_ref[:]
    agg = p0_ref[0] + p0_ref[1] + p1_ref[0] + p1_ref[1]
    h = (jnp.dot(nf, w1n_ref[:], preferred_element_type=jnp.float32)
         + jnp.dot(agg, w1a_ref[:], preferred_element_type=jnp.float32)
         + b1_ref[:])
    h = jnp.maximum(h, 0.0)
    h = jnp.dot(h, w2_ref[:], preferred_element_type=jnp.float32) + b2_ref[:]
    h = jnp.maximum(h, 0.0)
    h = jnp.dot(h, w3_ref[:], preferred_element_type=jnp.float32) + b3_ref[:]
    mu = jnp.mean(h, axis=1, keepdims=True)
    d = h - mu
    var = jnp.mean(d * d, axis=1, keepdims=True)
    y = d * lax.rsqrt(var + 1e-5) * g_ref[:] + bt_ref[:]
    out_ref[:] = y + nf


def _node_mlp(nf, p0, p1, w1n, w1a, b1, w2, b2, w3, b3, g, bt):
    BN = 1000
    grid = (N // BN,)
    blk = lambda i: (i, 0)
    pspec = pl.BlockSpec((NC, BN, D), lambda i: (0, i, 0))
    wspec = pl.BlockSpec((D, D), lambda i: (0, 0))
    vspec = pl.BlockSpec((1, D), lambda i: (0, 0))
    return pl.pallas_call(
        _node_mlp_kernel,
        grid=grid,
        in_specs=[
            pl.BlockSpec((BN, D), blk),
            pspec, pspec,
            wspec, wspec, vspec, wspec, vspec, wspec, vspec, vspec, vspec,
        ],
        out_specs=pl.BlockSpec((BN, D), blk),
        out_shape=jax.ShapeDtypeStruct((N, D), jnp.float32),
    )(nf, p0, p1, w1n, w1a, b1, w2, b2, w3, b3, g, bt)


def _gather(ps, pr, snd, rcv):
    mesh = plsc.VectorSubcoreMesh(core_axis_name="c", subcore_axis_name="s")
    epc = CE // NS

    @functools.partial(
        pl.kernel,
        mesh=mesh,
        out_type=(jax.ShapeDtypeStruct((CE, D), jnp.float32),
                  jax.ShapeDtypeStruct((CE, D), jnp.float32)),
        scratch_types=[
            pltpu.VMEM((C,), jnp.int32),
            pltpu.VMEM((C,), jnp.int32),
            pltpu.VMEM((C, D), jnp.float32),
            pltpu.VMEM((C, D), jnp.float32),
            pltpu.VMEM_SHARED((N, D), jnp.float32),
            pltpu.SemaphoreType.DMA,
            pltpu.SemaphoreType.DMA,
        ],
    )
    def k(ps_h, pr_h, snd_h, rcv_h, gs_h, gr_h, idxa, idxb, row0, row1, tab,
          sem0, sem1):
        c = lax.axis_index("c")
        s = lax.axis_index("s")

        def run(tab_h, idx_h, out_h):
            r0 = s * ROWS_PER_TILE

            @pl.when(s < NS - 1)
            def _():
                pltpu.sync_copy(tab_h.at[pl.ds(r0, ROWS_PER_TILE)],
                                tab.at[pl.ds(r0, ROWS_PER_TILE)])

            @pl.when(s == NS - 1)
            def _():
                nrem = N - (NS - 1) * ROWS_PER_TILE
                pltpu.sync_copy(tab_h.at[pl.ds(r0, nrem)],
                                tab.at[pl.ds(r0, nrem)])

            plsc.subcore_barrier()

            base = s * epc

            def body(i, carry):
                off = base + 2 * i * C
                pltpu.sync_copy(idx_h.at[pl.ds(off, C)], idxa)
                ca = pltpu.async_copy(tab.at[idxa], row0, sem0)
                pltpu.sync_copy(idx_h.at[pl.ds(off + C, C)], idxb)
                cb = pltpu.async_copy(tab.at[idxb], row1, sem1)
                ca.wait()
                pltpu.sync_copy(row0, out_h.at[pl.ds(off, C)])
                cb.wait()
                pltpu.sync_copy(row1, out_h.at[pl.ds(off + C, C)])
                return carry

            lax.fori_loop(0, epc // (2 * C), body, 0)

        @pl.when(c == 0)
        def _():
            run(ps_h, snd_h, gs_h)

        @pl.when(c == 1)
        def _():
            run(pr_h, rcv_h, gr_h)

    return k(ps, pr, snd, rcv)


def _scatter_multi(ln_chunks, rcv, zeros, k0):
    mesh = plsc.VectorSubcoreMesh(core_axis_name="c", subcore_axis_name="s")
    nk = len(ln_chunks)
    epc = CE // NC
    ept = epc // NS

    @functools.partial(
        pl.kernel,
        mesh=mesh,
        out_type=jax.ShapeDtypeStruct((NC, N, D), jnp.float32),
        scratch_types=[
            pltpu.VMEM((C,), jnp.int32),
            pltpu.VMEM((C,), jnp.int32),
            pltpu.VMEM((C, D), jnp.float32),
            pltpu.VMEM((C, D), jnp.float32),
            pltpu.VMEM_SHARED((N, D), jnp.float32),
            pltpu.SemaphoreType.DMA,
            pltpu.SemaphoreType.DMA,
        ],
    )
    def k(*refs):
        e_hs = refs[:nk]
        rcv_h, z_h, out_h, idxa, idxb, rows0, rows1, acc, sem0, sem1 = refs[nk:]
        c = lax.axis_index("c")
        s = lax.axis_index("s")
        r0 = s * ROWS_PER_TILE
        @pl.when(s < NS - 1)
        def _():
            pltpu.sync_copy(z_h.at[pl.ds(r0, ROWS_PER_TILE)],
                            acc.at[pl.ds(r0, ROWS_PER_TILE)])

        @pl.when(s == NS - 1)
        def _():
            pltpu.sync_copy(z_h.at[pl.ds(r0, N - (NS - 1) * ROWS_PER_TILE)],
                            acc.at[pl.ds(r0, N - (NS - 1) * ROWS_PER_TILE)])

        plsc.subcore_barrier()

        base = c * epc + s * ept

        for j in range(nk):
            e_h = e_hs[j]
            goff = (k0 + j) * CE + base

            def body(i, carry, e_h=e_h, goff=goff):
                off = base + 2 * i * C
                gi = goff + 2 * i * C
                pltpu.sync_copy(rcv_h.at[pl.ds(gi, C)], idxa)
                ca = pltpu.async_copy(e_h.at[pl.ds(off, C)], rows0, sem0)
                pltpu.sync_copy(rcv_h.at[pl.ds(gi + C, C)], idxb)
                cb = pltpu.async_copy(e_h.at[pl.ds(off + C, C)], rows1, sem1)
                ca.wait()
                pltpu.sync_copy(rows0, acc.at[idxa], add=True)
                cb.wait()
                pltpu.sync_copy(rows1, acc.at[idxb], add=True)
                return carry

            lax.fori_loop(0, ept // (2 * C), body, 0)
            if ept % (2 * C) != 0:
                toff = base + (ept // (2 * C)) * 2 * C
                gtoff = goff + (ept // (2 * C)) * 2 * C
                pltpu.sync_copy(rcv_h.at[pl.ds(gtoff, C)], idxa)
                pltpu.sync_copy(e_h.at[pl.ds(toff, C)], rows0)
                pltpu.sync_copy(rows0, acc.at[idxa], add=True)

        plsc.subcore_barrier()

        @pl.when(s < NS - 1)
        def _():
            pltpu.sync_copy(acc.at[pl.ds(r0, ROWS_PER_TILE)],
                            out_h.at[c, pl.ds(r0, ROWS_PER_TILE)])

        @pl.when(s == NS - 1)
        def _():
            pltpu.sync_copy(acc.at[pl.ds(r0, N - (NS - 1) * ROWS_PER_TILE)],
                            out_h.at[c, pl.ds(r0, N - (NS - 1) * ROWS_PER_TILE)])

    return k(*ln_chunks, rcv, zeros)


def kernel(node_features, edge_features, senders, receivers,
           ew1, eb1, ew2, eb2, ew3, eb3, eg, ebt,
           nw1, nb1, nw2, nb2, nw3, nb3, ng, nbt):
    w1s, w1r, w1e = ew1[:D], ew1[D:2 * D], ew1[2 * D:]
    w1n, w1a = nw1[:D], nw1[D:]
    eb1r = eb1.reshape(1, D)
    eb2r = eb2.reshape(1, D)
    eb3r = eb3.reshape(1, D)
    egr = eg.reshape(1, D)
    ebtr = ebt.reshape(1, D)
    nb1r = nb1.reshape(1, D)
    nb2r = nb2.reshape(1, D)
    nb3r = nb3.reshape(1, D)
    ngr = ng.reshape(1, D)
    nbtr = nbt.reshape(1, D)

    ps, pr = _project(node_features, w1s, w1r)
    zeros = jnp.zeros((N, D), jnp.float32)

    lns = []
    e_acc = None
    for k in range(K):
        lo = k * CE
        snd_k = lax.slice(senders, (lo,), (lo + CE,))
        rcv_k = lax.slice(receivers, (lo,), (lo + CE,))
        gs_k, gr_k = _gather(ps, pr, snd_k, rcv_k)
        ln_k, e_acc = _edge_mlp_chunk(k, gs_k, gr_k, edge_features,
                                      w1e, eb1r, ew2, eb2r, ew3, eb3r,
                                      egr, ebtr, e_acc)
        lns.append(ln_k)

    p0 = _scatter_multi(lns[:3], receivers, zeros, 0)
    p1 = _scatter_multi(lns[3:], receivers, zeros, 3)

    new_n = _node_mlp(node_features, p0, p1, w1n, w1a, nb1r,
                      nw2, nb2r, nw3, nb3r, ngr, nbtr)
    return (new_n, e_acc)
